# Initial kernel scaffold; baseline (speedup 1.0000x reference)
import functools

import jax
import jax.numpy as jnp
from jax import lax
from jax.experimental import pallas as pl
from jax.experimental.pallas import tpu as pltpu
from jax.experimental.pallas import tpu_sc as plsc

N = 10000
E = 320000
D_IN = 128
H = 128
C = 64

NC = 2
NS = 16
NW = NC * NS
E_PER = E // NW
EB = 80
NCHUNK = E_PER // EB
RPT = N // NS
DEGW = 16
ZR = 125

_mesh = plsc.VectorSubcoreMesh(core_axis_name="c", subcore_axis_name="s")


def _fill_zeros(ref, rows, width):
    def body(i, _):
        def inner(j, _):
            ref[i, pl.ds(j * 16, 16)] = jnp.zeros((16,), jnp.float32)
            return 0
        return lax.fori_loop(0, width // 16, inner, 0)
    lax.fori_loop(0, rows, body, 0)


def _deg_body(src_hbm, dst_hbm, out_hbm, in_hbm, idx_v, ones_v, zero_v,
              acc_out, acc_in):
    cid = lax.axis_index("c")
    sid = lax.axis_index("s")
    wid = cid * NS + sid

    _fill_zeros(zero_v, ZR, DEGW)

    def fill_ones(i, _):
        ones_v[i, pl.ds(0, 16)] = jnp.ones((16,), jnp.float32)
        return 0
    lax.fori_loop(0, EB, fill_ones, 0)

    r0 = sid * RPT
    for k in range(RPT // ZR):
        pltpu.sync_copy(zero_v, acc_out.at[pl.ds(r0 + k * ZR, ZR)])
        pltpu.sync_copy(zero_v, acc_in.at[pl.ds(r0 + k * ZR, ZR)])
    plsc.subcore_barrier()

    def body(i, _):
        base = wid * E_PER + i * EB
        pltpu.sync_copy(src_hbm.at[pl.ds(base, EB)], idx_v)
        pltpu.sync_copy(ones_v, acc_out.at[idx_v], add=True)
        pltpu.sync_copy(dst_hbm.at[pl.ds(base, EB)], idx_v)
        pltpu.sync_copy(ones_v, acc_in.at[idx_v], add=True)
        return 0
    lax.fori_loop(0, NCHUNK, body, 0)
    plsc.subcore_barrier()

    pltpu.sync_copy(acc_out.at[pl.ds(r0, RPT)], out_hbm.at[cid, pl.ds(r0, RPT)])
    pltpu.sync_copy(acc_in.at[pl.ds(r0, RPT)], in_hbm.at[cid, pl.ds(r0, RPT)])


_deg_kernel = pl.kernel(
    _deg_body,
    out_type=[jax.ShapeDtypeStruct((NC, N, DEGW), jnp.float32),
              jax.ShapeDtypeStruct((NC, N, DEGW), jnp.float32)],
    mesh=_mesh,
    scratch_types=[
        pltpu.VMEM((EB,), jnp.int32),
        pltpu.VMEM((EB, DEGW), jnp.float32),
        pltpu.VMEM((ZR, DEGW), jnp.float32),
        pltpu.VMEM_SHARED((N, DEGW), jnp.float32),
        pltpu.VMEM_SHARED((N, DEGW), jnp.float32),
    ],
)


def _agg_body(h_hbm, src_hbm, dst_hbm, out_hbm, src_v, dst_v, rows_v, zero_v,
              acc, sem, *, d):
    cid = lax.axis_index("c")
    sid = lax.axis_index("s")
    wid = cid * NS + sid

    _fill_zeros(zero_v, ZR, d)
    r0 = sid * RPT
    for k in range(RPT // ZR):
        pltpu.sync_copy(zero_v, acc.at[pl.ds(r0 + k * ZR, ZR)])
    plsc.subcore_barrier()

    def body(i, _):
        base = wid * E_PER + i * EB
        pltpu.sync_copy(src_hbm.at[pl.ds(base, EB)], src_v)
        pltpu.sync_copy(dst_hbm.at[pl.ds(base, EB)], dst_v)
        pltpu.async_copy(h_hbm.at[src_v], rows_v, sem).wait()
        pltpu.sync_copy(rows_v, acc.at[dst_v], add=True)
        return 0
    lax.fori_loop(0, NCHUNK, body, 0)
    plsc.subcore_barrier()

    pltpu.sync_copy(acc.at[pl.ds(r0, RPT)], out_hbm.at[cid, pl.ds(r0, RPT)])


def _make_agg(d):
    return pl.kernel(
        functools.partial(_agg_body, d=d),
        out_type=jax.ShapeDtypeStruct((NC, N, d), jnp.float32),
        mesh=_mesh,
        scratch_types=[
            pltpu.VMEM((EB,), jnp.int32),
            pltpu.VMEM((EB,), jnp.int32),
            pltpu.VMEM((EB, d), jnp.float32),
            pltpu.VMEM((ZR, d), jnp.float32),
            pltpu.VMEM_SHARED((N, d), jnp.float32),
            pltpu.SemaphoreType.DMA,
        ],
    )


_agg_h = _make_agg(H)
_agg_c = _make_agg(C)


R = 1000


def _scale_from_counts(cnt_ref):
    cnt = cnt_ref[0, :, :] + cnt_ref[1, :, :]
    return lax.rsqrt(jnp.maximum(cnt[:, 0:1], 1.0))


def _l1_body(x_ref, co_ref, w_ref, o_ref):
    s = _scale_from_counts(co_ref)
    o_ref[...] = jnp.dot(x_ref[...] * s, w_ref[...],
                         preferred_element_type=jnp.float32)


def _mid_body(agg_ref, ci_ref, co_ref, b_ref, w_ref, o_ref):
    a = agg_ref[0, :, :] + agg_ref[1, :, :]
    si = _scale_from_counts(ci_ref)
    t = jnp.maximum(a * si + b_ref[...], 0.0)
    so = _scale_from_counts(co_ref)
    o_ref[...] = jnp.dot(t * so, w_ref[...],
                         preferred_element_type=jnp.float32)


def _final_body(agg_ref, ci_ref, b_ref, o_ref):
    a = agg_ref[0, :, :] + agg_ref[1, :, :]
    si = _scale_from_counts(ci_ref)
    o_ref[...] = a * si + b_ref[...]


def _cnt_spec():
    return pl.BlockSpec((NC, R, DEGW), lambda i: (0, i, 0))


def _l1(x, cnt_out, w):
    return pl.pallas_call(
        _l1_body,
        grid=(N // R,),
        in_specs=[pl.BlockSpec((R, D_IN), lambda i: (i, 0)),
                  _cnt_spec(),
                  pl.BlockSpec((D_IN, H), lambda i: (0, 0))],
        out_specs=pl.BlockSpec((R, H), lambda i: (i, 0)),
        out_shape=jax.ShapeDtypeStruct((N, H), jnp.float32),
    )(x, cnt_out, w)


def _mid(agg, cnt_in, cnt_out, b, w, dout):
    return pl.pallas_call(
        _mid_body,
        grid=(N // R,),
        in_specs=[pl.BlockSpec((NC, R, H), lambda i: (0, i, 0)),
                  _cnt_spec(), _cnt_spec(),
                  pl.BlockSpec((1, H), lambda i: (0, 0)),
                  pl.BlockSpec((H, dout), lambda i: (0, 0))],
        out_specs=pl.BlockSpec((R, dout), lambda i: (i, 0)),
        out_shape=jax.ShapeDtypeStruct((N, dout), jnp.float32),
    )(agg, cnt_in, cnt_out, b.reshape(1, H), w)


def _final(agg, cnt_in, b):
    return pl.pallas_call(
        _final_body,
        grid=(N // R,),
        in_specs=[pl.BlockSpec((NC, R, C), lambda i: (0, i, 0)),
                  _cnt_spec(),
                  pl.BlockSpec((1, C), lambda i: (0, 0))],
        out_specs=pl.BlockSpec((R, C), lambda i: (i, 0)),
        out_shape=jax.ShapeDtypeStruct((N, C), jnp.float32),
    )(agg, cnt_in, b.reshape(1, C))


def kernel(features, edge_index, W1, b1, W2, b2, W3, b3):
    src = edge_index[0]
    dst = edge_index[1]
    cnt_out, cnt_in = _deg_kernel(src, dst)
    h1 = _l1(features, cnt_out, W1)
    agg1 = _agg_h(h1, src, dst)
    h2 = _mid(agg1, cnt_in, cnt_out, b1, W2, H)
    agg2 = _agg_h(h2, src, dst)
    h3 = _mid(agg2, cnt_in, cnt_out, b2, W3)
    agg3 = _agg_c(h3, src, dst)
    return _final(agg3, cnt_in, b3)


# speedup vs baseline: 3.9363x; 3.9363x over previous
import functools

import jax
import jax.numpy as jnp
from jax import lax
from jax.experimental import pallas as pl
from jax.experimental.pallas import tpu as pltpu
from jax.experimental.pallas import tpu_sc as plsc

N = 10000
E = 320000
D_IN = 128
H = 128
C = 64

NC = 2
NS = 16
NW = NC * NS
E_PER = E // NW
EB = 80
NCHUNK = E_PER // EB
RB = 624
REM = N - RB * NS
ZR = 208

_mesh = plsc.VectorSubcoreMesh(core_axis_name="c", subcore_axis_name="s")


def _fill_zeros(ref, rows, width):
    def body(i, _):
        def inner(j, _):
            ref[i, pl.ds(j * 16, 16)] = jnp.zeros((16,), jnp.float32)
            return 0
        return lax.fori_loop(0, width // 16, inner, 0)
    lax.fori_loop(0, rows, body, 0)


def _zero_acc(zero_v, acc, sid, width_rows=ZR):
    r0 = sid * RB
    for k in range(RB // ZR):
        pltpu.sync_copy(zero_v, acc.at[pl.ds(r0 + k * ZR, ZR)])

    @pl.when(sid == NS - 1)
    def _zero_tail():
        pltpu.sync_copy(zero_v.at[pl.ds(0, REM)], acc.at[pl.ds(RB * NS, REM)])


def _writeback(acc, out_hbm, cid, sid):
    r0 = sid * RB
    pltpu.sync_copy(acc.at[pl.ds(r0, RB)], out_hbm.at[cid, pl.ds(r0, RB)])

    @pl.when(sid == NS - 1)
    def _write_tail():
        pltpu.sync_copy(acc.at[pl.ds(RB * NS, REM)],
                        out_hbm.at[cid, pl.ds(RB * NS, REM)])


def _deg_body(src_hbm, dst_hbm, out_hbm, in_hbm, idx_v, ones_v, zero_v, acc):
    cid = lax.axis_index("c")
    sid = lax.axis_index("s")
    wid = cid * NS + sid

    _fill_zeros(zero_v, ZR, H)

    def fill_ones(i, _):
        def inner(j, _):
            ones_v[i, pl.ds(j * 16, 16)] = jnp.ones((16,), jnp.float32)
            return 0
        return lax.fori_loop(0, H // 16, inner, 0)
    lax.fori_loop(0, EB, fill_ones, 0)

    for idx_hbm, o_hbm in ((src_hbm, out_hbm), (dst_hbm, in_hbm)):
        _zero_acc(zero_v, acc, sid)
        plsc.subcore_barrier()

        def body(i, _):
            base = wid * E_PER + i * EB
            pltpu.sync_copy(idx_hbm.at[pl.ds(base, EB)], idx_v)
            pltpu.sync_copy(ones_v, acc.at[idx_v], add=True)
            return 0
        lax.fori_loop(0, NCHUNK, body, 0)
        plsc.subcore_barrier()

        _writeback(acc, o_hbm, cid, sid)
        plsc.subcore_barrier()


_deg_kernel = pl.kernel(
    _deg_body,
    out_type=[jax.ShapeDtypeStruct((NC, N, H), jnp.float32),
              jax.ShapeDtypeStruct((NC, N, H), jnp.float32)],
    mesh=_mesh,
    scratch_types=[
        pltpu.VMEM((EB,), jnp.int32),
        pltpu.VMEM((EB, H), jnp.float32),
        pltpu.VMEM((ZR, H), jnp.float32),
        pltpu.VMEM_SHARED((N, H), jnp.float32),
    ],
)


def _agg_body(h_hbm, src_hbm, dst_hbm, out_hbm, src_v, dst_v, rows_v, zero_v,
              acc, sem, *, d):
    cid = lax.axis_index("c")
    sid = lax.axis_index("s")
    wid = cid * NS + sid

    _fill_zeros(zero_v, ZR, d)
    _zero_acc(zero_v, acc, sid)
    plsc.subcore_barrier()

    def body(i, _):
        base = wid * E_PER + i * EB
        pltpu.sync_copy(src_hbm.at[pl.ds(base, EB)], src_v)
        pltpu.sync_copy(dst_hbm.at[pl.ds(base, EB)], dst_v)
        pltpu.async_copy(h_hbm.at[src_v], rows_v, sem).wait()
        pltpu.sync_copy(rows_v, acc.at[dst_v], add=True)
        return 0
    lax.fori_loop(0, NCHUNK, body, 0)
    plsc.subcore_barrier()
    _writeback(acc, out_hbm, cid, sid)


def _make_agg(d):
    return pl.kernel(
        functools.partial(_agg_body, d=d),
        out_type=jax.ShapeDtypeStruct((NC, N, d), jnp.float32),
        mesh=_mesh,
        scratch_types=[
            pltpu.VMEM((EB,), jnp.int32),
            pltpu.VMEM((EB,), jnp.int32),
            pltpu.VMEM((EB, d), jnp.float32),
            pltpu.VMEM((ZR, d), jnp.float32),
            pltpu.VMEM_SHARED((N, d), jnp.float32),
            pltpu.SemaphoreType.DMA,
        ],
    )


_agg_h = _make_agg(H)


R = 1000


def _scale_from_counts(cnt_ref):
    cnt = cnt_ref[0, :, :] + cnt_ref[1, :, :]
    return lax.rsqrt(jnp.maximum(cnt[:, 0:1], 1.0))


def _l1_body(x_ref, co_ref, w_ref, o_ref):
    s = _scale_from_counts(co_ref)
    o_ref[...] = jnp.dot(x_ref[...] * s, w_ref[...],
                         preferred_element_type=jnp.float32)


def _mid_body(agg_ref, ci_ref, co_ref, b_ref, w_ref, o_ref):
    a = agg_ref[0, :, :] + agg_ref[1, :, :]
    si = _scale_from_counts(ci_ref)
    t = jnp.maximum(a * si + b_ref[...], 0.0)
    so = _scale_from_counts(co_ref)
    o_ref[...] = jnp.dot(t * so, w_ref[...],
                         preferred_element_type=jnp.float32)


def _final_body(agg_ref, ci_ref, b_ref, o_ref):
    a = agg_ref[0, :, :C] + agg_ref[1, :, :C]
    si = _scale_from_counts(ci_ref)
    o_ref[...] = a * si + b_ref[...]


def _cnt_spec():
    return pl.BlockSpec((NC, R, H), lambda i: (0, i, 0))


def _l1(x, cnt_out, w):
    return pl.pallas_call(
        _l1_body,
        grid=(N // R,),
        in_specs=[pl.BlockSpec((R, D_IN), lambda i: (i, 0)),
                  _cnt_spec(),
                  pl.BlockSpec((D_IN, H), lambda i: (0, 0))],
        out_specs=pl.BlockSpec((R, H), lambda i: (i, 0)),
        out_shape=jax.ShapeDtypeStruct((N, H), jnp.float32),
    )(x, cnt_out, w)


def _mid(agg, cnt_in, cnt_out, b, w, dout):
    return pl.pallas_call(
        _mid_body,
        grid=(N // R,),
        in_specs=[pl.BlockSpec((NC, R, H), lambda i: (0, i, 0)),
                  _cnt_spec(), _cnt_spec(),
                  pl.BlockSpec((1, H), lambda i: (0, 0)),
                  pl.BlockSpec((H, dout), lambda i: (0, 0))],
        out_specs=pl.BlockSpec((R, dout), lambda i: (i, 0)),
        out_shape=jax.ShapeDtypeStruct((N, dout), jnp.float32),
    )(agg, cnt_in, cnt_out, b.reshape(1, H), w)


def _final(agg, cnt_in, b):
    return pl.pallas_call(
        _final_body,
        grid=(N // R,),
        in_specs=[pl.BlockSpec((NC, R, H), lambda i: (0, i, 0)),
                  _cnt_spec(),
                  pl.BlockSpec((1, C), lambda i: (0, 0))],
        out_specs=pl.BlockSpec((R, C), lambda i: (i, 0)),
        out_shape=jax.ShapeDtypeStruct((N, C), jnp.float32),
    )(agg, cnt_in, b.reshape(1, C))


def kernel(features, edge_index, W1, b1, W2, b2, W3, b3):
    src = edge_index[0]
    dst = edge_index[1]
    cnt_out, cnt_in = _deg_kernel(src, dst)
    h1 = _l1(features, cnt_out, W1)
    agg1 = _agg_h(h1, src, dst)
    h2 = _mid(agg1, cnt_in, cnt_out, b1, W2, H)
    agg2 = _agg_h(h2, src, dst)
    W3p = jnp.pad(W3, ((0, 0), (0, H - C)))
    h3 = _mid(agg2, cnt_in, cnt_out, b2, W3p, H)
    agg3 = _agg_h(h3, src, dst)
    return _final(agg3, cnt_in, b3)


# speedup vs baseline: 7.0041x; 1.7794x over previous
import functools

import jax
import jax.numpy as jnp
from jax import lax
from jax.experimental import pallas as pl
from jax.experimental.pallas import tpu as pltpu
from jax.experimental.pallas import tpu_sc as plsc

N = 10000
E = 320000
D_IN = 128
H = 128
C = 64

NC = 2
NS = 16
NW = NC * NS
E_PER = E // NW
EB = 80
NCHUNK = E_PER // EB
RB = 624
REM = N - RB * NS
ZR = 208

_mesh = plsc.VectorSubcoreMesh(core_axis_name="c", subcore_axis_name="s")


def _fill_zeros(ref, rows, width):
    def body(i, _):
        def inner(j, _):
            ref[i, pl.ds(j * 16, 16)] = jnp.zeros((16,), jnp.float32)
            return 0
        return lax.fori_loop(0, width // 16, inner, 0)
    lax.fori_loop(0, rows, body, 0)


def _zero_acc(zero_v, acc, sid, width_rows=ZR):
    r0 = sid * RB
    for k in range(RB // ZR):
        pltpu.sync_copy(zero_v, acc.at[pl.ds(r0 + k * ZR, ZR)])

    @pl.when(sid == NS - 1)
    def _zero_tail():
        pltpu.sync_copy(zero_v.at[pl.ds(0, REM)], acc.at[pl.ds(RB * NS, REM)])


def _writeback(acc, out_hbm, cid, sid):
    r0 = sid * RB
    pltpu.sync_copy(acc.at[pl.ds(r0, RB)], out_hbm.at[cid, pl.ds(r0, RB)])

    @pl.when(sid == NS - 1)
    def _write_tail():
        pltpu.sync_copy(acc.at[pl.ds(RB * NS, REM)],
                        out_hbm.at[cid, pl.ds(RB * NS, REM)])


NBUF = 4
NFULL = NCHUNK // NBUF


def _deg_body(src_hbm, dst_hbm, out_hbm, in_hbm, i0, i1, i2, i3, ones_v,
              zero_v, acc, s0, s1, s2, s3):
    cid = lax.axis_index("c")
    sid = lax.axis_index("s")
    wid = cid * NS + sid
    ibufs = (i0, i1, i2, i3)
    sems = (s0, s1, s2, s3)

    _fill_zeros(zero_v, ZR, H)

    def fill_ones(i, _):
        def inner(j, _):
            ones_v[i, pl.ds(j * 16, 16)] = jnp.ones((16,), jnp.float32)
            return 0
        return lax.fori_loop(0, H // 16, inner, 0)
    lax.fori_loop(0, EB, fill_ones, 0)

    for idx_hbm, o_hbm in ((src_hbm, out_hbm), (dst_hbm, in_hbm)):
        _zero_acc(zero_v, acc, sid)
        plsc.subcore_barrier()

        def body(k, _):
            loads = [pltpu.async_copy(
                idx_hbm.at[pl.ds(wid * E_PER + (k * NBUF + b) * EB, EB)],
                ibufs[b], sems[b]) for b in range(NBUF)]
            scats = []
            for b in range(NBUF):
                loads[b].wait()
                scats.append(pltpu.async_copy(ones_v, acc.at[ibufs[b]],
                                              sems[b], add=True))
            for sc in scats:
                sc.wait()
            return 0
        lax.fori_loop(0, NFULL, body, 0)
        for i in range(NFULL * NBUF, NCHUNK):
            pltpu.sync_copy(idx_hbm.at[pl.ds(wid * E_PER + i * EB, EB)], i0)
            pltpu.sync_copy(ones_v, acc.at[i0], add=True)
        plsc.subcore_barrier()

        _writeback(acc, o_hbm, cid, sid)
        plsc.subcore_barrier()


_deg_kernel = pl.kernel(
    _deg_body,
    out_type=[jax.ShapeDtypeStruct((NC, N, H), jnp.float32),
              jax.ShapeDtypeStruct((NC, N, H), jnp.float32)],
    mesh=_mesh,
    scratch_types=(
        [pltpu.VMEM((EB,), jnp.int32)] * NBUF
        + [pltpu.VMEM((EB, H), jnp.float32),
           pltpu.VMEM((ZR, H), jnp.float32),
           pltpu.VMEM_SHARED((N, H), jnp.float32)]
        + [pltpu.SemaphoreType.DMA] * NBUF
    ),
)


def _agg_body(h_hbm, src_hbm, dst_hbm, out_hbm, i0, i1, i2, i3, r0, r1, r2,
              r3, acc, s0, s1, s2, s3, *, d):
    cid = lax.axis_index("c")
    sid = lax.axis_index("s")
    wid = cid * NS + sid
    ibufs = (i0, i1, i2, i3)
    rbufs = (r0, r1, r2, r3)
    sems = (s0, s1, s2, s3)

    _fill_zeros(r0, EB, d)
    zb = sid * RB
    for k in range(RB // EB):
        pltpu.sync_copy(r0, acc.at[pl.ds(zb + k * EB, EB)])
    pltpu.sync_copy(r0.at[pl.ds(0, RB % EB)],
                    acc.at[pl.ds(zb + RB - RB % EB, RB % EB)])

    @pl.when(sid == NS - 1)
    def _zero_tail():
        pltpu.sync_copy(r0.at[pl.ds(0, REM)], acc.at[pl.ds(RB * NS, REM)])
    plsc.subcore_barrier()

    def body(k, _):
        loads = []
        for b in range(NBUF):
            base = wid * E_PER + (k * NBUF + b) * EB
            loads.append((
                pltpu.async_copy(src_hbm.at[pl.ds(base, EB)],
                                 ibufs[b].at[0], sems[b]),
                pltpu.async_copy(dst_hbm.at[pl.ds(base, EB)],
                                 ibufs[b].at[1], sems[b])))
        gathers = []
        for b in range(NBUF):
            loads[b][0].wait()
            loads[b][1].wait()
            gathers.append(pltpu.async_copy(h_hbm.at[ibufs[b].at[0]],
                                            rbufs[b], sems[b]))
        scats = []
        for b in range(NBUF):
            gathers[b].wait()
            scats.append(pltpu.async_copy(rbufs[b], acc.at[ibufs[b].at[1]],
                                          sems[b], add=True))
        for sc in scats:
            sc.wait()
        return 0
    lax.fori_loop(0, NFULL, body, 0)
    for i in range(NFULL * NBUF, NCHUNK):
        base = wid * E_PER + i * EB
        pltpu.sync_copy(src_hbm.at[pl.ds(base, EB)], i0.at[0])
        pltpu.sync_copy(dst_hbm.at[pl.ds(base, EB)], i0.at[1])
        pltpu.async_copy(h_hbm.at[i0.at[0]], r0, s0).wait()
        pltpu.sync_copy(r0, acc.at[i0.at[1]], add=True)
    plsc.subcore_barrier()
    _writeback(acc, out_hbm, cid, sid)


def _make_agg(d):
    return pl.kernel(
        functools.partial(_agg_body, d=d),
        out_type=jax.ShapeDtypeStruct((NC, N, d), jnp.float32),
        mesh=_mesh,
        scratch_types=(
            [pltpu.VMEM((2, EB), jnp.int32)] * NBUF
            + [pltpu.VMEM((EB, d), jnp.float32)] * NBUF
            + [pltpu.VMEM_SHARED((N, d), jnp.float32)]
            + [pltpu.SemaphoreType.DMA] * NBUF
        ),
    )


_agg_h = _make_agg(H)


R = 1000


def _scale_from_counts(cnt_ref):
    cnt = cnt_ref[0, :, :] + cnt_ref[1, :, :]
    return lax.rsqrt(jnp.maximum(cnt[:, 0:1], 1.0))


def _l1_body(x_ref, co_ref, w_ref, o_ref):
    s = _scale_from_counts(co_ref)
    o_ref[...] = jnp.dot(x_ref[...] * s, w_ref[...],
                         preferred_element_type=jnp.float32)


def _mid_body(agg_ref, ci_ref, co_ref, b_ref, w_ref, o_ref):
    a = agg_ref[0, :, :] + agg_ref[1, :, :]
    si = _scale_from_counts(ci_ref)
    t = jnp.maximum(a * si + b_ref[...], 0.0)
    so = _scale_from_counts(co_ref)
    o_ref[...] = jnp.dot(t * so, w_ref[...],
                         preferred_element_type=jnp.float32)


def _final_body(agg_ref, ci_ref, b_ref, o_ref):
    a = agg_ref[0, :, :C] + agg_ref[1, :, :C]
    si = _scale_from_counts(ci_ref)
    o_ref[...] = a * si + b_ref[...]


def _cnt_spec():
    return pl.BlockSpec((NC, R, H), lambda i: (0, i, 0))


def _l1(x, cnt_out, w):
    return pl.pallas_call(
        _l1_body,
        grid=(N // R,),
        in_specs=[pl.BlockSpec((R, D_IN), lambda i: (i, 0)),
                  _cnt_spec(),
                  pl.BlockSpec((D_IN, H), lambda i: (0, 0))],
        out_specs=pl.BlockSpec((R, H), lambda i: (i, 0)),
        out_shape=jax.ShapeDtypeStruct((N, H), jnp.float32),
    )(x, cnt_out, w)


def _mid(agg, cnt_in, cnt_out, b, w, dout):
    return pl.pallas_call(
        _mid_body,
        grid=(N // R,),
        in_specs=[pl.BlockSpec((NC, R, H), lambda i: (0, i, 0)),
                  _cnt_spec(), _cnt_spec(),
                  pl.BlockSpec((1, H), lambda i: (0, 0)),
                  pl.BlockSpec((H, dout), lambda i: (0, 0))],
        out_specs=pl.BlockSpec((R, dout), lambda i: (i, 0)),
        out_shape=jax.ShapeDtypeStruct((N, dout), jnp.float32),
    )(agg, cnt_in, cnt_out, b.reshape(1, H), w)


def _final(agg, cnt_in, b):
    return pl.pallas_call(
        _final_body,
        grid=(N // R,),
        in_specs=[pl.BlockSpec((NC, R, H), lambda i: (0, i, 0)),
                  _cnt_spec(),
                  pl.BlockSpec((1, C), lambda i: (0, 0))],
        out_specs=pl.BlockSpec((R, C), lambda i: (i, 0)),
        out_shape=jax.ShapeDtypeStruct((N, C), jnp.float32),
    )(agg, cnt_in, b.reshape(1, C))


def kernel(features, edge_index, W1, b1, W2, b2, W3, b3):
    src = edge_index[0]
    dst = edge_index[1]
    cnt_out, cnt_in = _deg_kernel(src, dst)
    h1 = _l1(features, cnt_out, W1)
    agg1 = _agg_h(h1, src, dst)
    h2 = _mid(agg1, cnt_in, cnt_out, b1, W2, H)
    agg2 = _agg_h(h2, src, dst)
    W3p = jnp.pad(W3, ((0, 0), (0, H - C)))
    h3 = _mid(agg2, cnt_in, cnt_out, b2, W3p, H)
    agg3 = _agg_h(h3, src, dst)
    return _final(agg3, cnt_in, b3)
